# initial kernel scaffold (unmeasured)
import jax
import jax.numpy as jnp
from jax import lax
from jax.experimental import pallas as pl
from jax.experimental.pallas import tpu as pltpu


def kernel(A, B):
    M, K = A.shape
    _, N = B.shape

    def body(a_ref, b_ref, out_ref, acc_ref, send_sem, recv_sem):
        my_x = lax.axis_index("x")
        my_y = lax.axis_index("y")
        nbr = (1 - my_x, my_y)

        barrier = pltpu.get_barrier_semaphore()
        pl.semaphore_signal(
            barrier, inc=1, device_id=nbr, device_id_type=pl.DeviceIdType.MESH
        )
        pl.semaphore_wait(barrier, 1)

        out_ref[:, :] = jnp.dot(
            a_ref[:, :], b_ref[:, :], preferred_element_type=jnp.float32
        )

        rdma = pltpu.make_async_remote_copy(
            src_ref=out_ref,
            dst_ref=acc_ref,
            send_sem=send_sem,
            recv_sem=recv_sem,
            device_id=nbr,
            device_id_type=pl.DeviceIdType.MESH,
        )
        rdma.start()
        rdma.wait()

        out_ref[:, :] = out_ref[:, :] + acc_ref[:, :]

    return pl.pallas_call(
        body,
        out_shape=jax.ShapeDtypeStruct((M, N), jnp.float32),
        in_specs=[
            pl.BlockSpec(memory_space=pltpu.VMEM),
            pl.BlockSpec(memory_space=pltpu.VMEM),
        ],
        out_specs=pl.BlockSpec(memory_space=pltpu.VMEM),
        scratch_shapes=[
            pltpu.VMEM((M, N), jnp.float32),
            pltpu.SemaphoreType.DMA,
            pltpu.SemaphoreType.DMA,
        ],
        compiler_params=pltpu.CompilerParams(collective_id=0),
    )(A, B)


# baseline (device time: 218691 ns/iter reference)
import jax
import jax.numpy as jnp
from jax import lax
from jax.experimental import pallas as pl
from jax.experimental.pallas import tpu as pltpu


def kernel(A, B):
    M, K = A.shape
    _, N = B.shape

    def body(a_ref, b_ref, out_ref, acc_ref, send_sem, recv_sem):
        my_x = lax.axis_index("x")
        my_y = lax.axis_index("y")
        nbr = (1 - my_x, my_y)

        barrier = pltpu.get_barrier_semaphore()
        pl.semaphore_signal(
            barrier, inc=1, device_id=nbr, device_id_type=pl.DeviceIdType.MESH
        )
        pl.semaphore_wait(barrier, 1)

        for i in range(0, M, 512):
            out_ref[pl.ds(i, 512), :] = jnp.dot(
                a_ref[pl.ds(i, 512), :], b_ref[:, :],
                preferred_element_type=jnp.float32,
            )

        rdma = pltpu.make_async_remote_copy(
            src_ref=out_ref,
            dst_ref=acc_ref,
            send_sem=send_sem,
            recv_sem=recv_sem,
            device_id=nbr,
            device_id_type=pl.DeviceIdType.MESH,
        )
        rdma.start()
        rdma.wait()

        out_ref[:, :] = out_ref[:, :] + acc_ref[:, :]

    return pl.pallas_call(
        body,
        out_shape=jax.ShapeDtypeStruct((M, N), jnp.float32),
        in_specs=[
            pl.BlockSpec(memory_space=pltpu.VMEM),
            pl.BlockSpec(memory_space=pltpu.VMEM),
        ],
        out_specs=pl.BlockSpec(memory_space=pltpu.VMEM),
        scratch_shapes=[
            pltpu.VMEM((M, N), jnp.float32),
            pltpu.SemaphoreType.DMA,
            pltpu.SemaphoreType.DMA,
        ],
        compiler_params=pltpu.CompilerParams(
            collective_id=0,
            vmem_limit_bytes=100 * 1024 * 1024,
        ),
    )(A, B)


# device time: 132574 ns/iter; 1.6496x vs baseline; 1.6496x over previous
import jax
import jax.numpy as jnp
from jax import lax
from jax.experimental import pallas as pl
from jax.experimental.pallas import tpu as pltpu

C = 8


def kernel(A, B):
    M, K = A.shape
    _, N = B.shape
    half = M // 2
    chunk = half // C

    def body(a_ref, b_ref, out_ref, px_ref, sx_sem, rx_sem, sy_sem, ry_sem):
        my_x = lax.axis_index("x")
        my_y = lax.axis_index("y")
        nbr_x = (1 - my_x, my_y)
        nbr_y = (my_x, 1 - my_y)

        barrier = pltpu.get_barrier_semaphore()
        for nbr in (nbr_x, nbr_y):
            pl.semaphore_signal(
                barrier, inc=1, device_id=nbr,
                device_id_type=pl.DeviceIdType.MESH,
            )
        pl.semaphore_wait(barrier, 2)

        row0 = my_y * half

        rdma_x = []
        for c in range(C):
            r = row0 + c * chunk
            out_ref[pl.ds(r, chunk), :] = jnp.dot(
                a_ref[pl.ds(r, chunk), :], b_ref[:, :],
                preferred_element_type=jnp.float32,
            )
            rd = pltpu.make_async_remote_copy(
                src_ref=out_ref.at[pl.ds(r, chunk), :],
                dst_ref=px_ref.at[c],
                send_sem=sx_sem.at[c],
                recv_sem=rx_sem.at[c],
                device_id=nbr_x,
                device_id_type=pl.DeviceIdType.MESH,
            )
            rd.start()
            rdma_x.append(rd)

        rdma_y = []
        for c in range(C):
            r = row0 + c * chunk
            rdma_x[c].wait_recv()
            out_ref[pl.ds(r, chunk), :] = (
                out_ref[pl.ds(r, chunk), :] + px_ref[c]
            )
            rd = pltpu.make_async_remote_copy(
                src_ref=out_ref.at[pl.ds(r, chunk), :],
                dst_ref=out_ref.at[pl.ds(r, chunk), :],
                send_sem=sy_sem.at[c],
                recv_sem=ry_sem.at[c],
                device_id=nbr_y,
                device_id_type=pl.DeviceIdType.MESH,
            )
            rd.start()
            rdma_y.append(rd)

        for c in range(C):
            rdma_y[c].wait_recv()
        for c in range(C):
            rdma_x[c].wait_send()
            rdma_y[c].wait_send()

    return pl.pallas_call(
        body,
        out_shape=jax.ShapeDtypeStruct((M, N), jnp.float32),
        in_specs=[
            pl.BlockSpec(memory_space=pltpu.VMEM),
            pl.BlockSpec(memory_space=pltpu.VMEM),
        ],
        out_specs=pl.BlockSpec(memory_space=pltpu.VMEM),
        scratch_shapes=[
            pltpu.VMEM((C, chunk, N), jnp.float32),
            pltpu.SemaphoreType.DMA((C,)),
            pltpu.SemaphoreType.DMA((C,)),
            pltpu.SemaphoreType.DMA((C,)),
            pltpu.SemaphoreType.DMA((C,)),
        ],
        compiler_params=pltpu.CompilerParams(
            collective_id=0,
            vmem_limit_bytes=100 * 1024 * 1024,
        ),
    )(A, B)
